# Initial kernel scaffold; baseline (speedup 1.0000x reference)
import jax
import jax.numpy as jnp
from jax.experimental import pallas as pl


def kernel(h, edge_index, W_ih, W_hh, b_ih, b_hh, W_self, b_self, W_neigh, W1, b1, W2, b2):
    raise NotImplementedError("write your pallas kernel here")


# speedup vs baseline: 1.8676x; 1.8676x over previous
import functools
import jax
import jax.numpy as jnp
from jax.experimental import pallas as pl
from jax.experimental.pallas import tpu as pltpu


def _cell_body(t_ref, xt_ref, h_ref, c_ref, deg_ref, WihT_ref, WhhT_ref,
               b_ref, h_out, c_out):
    H = xt_ref.shape[1]
    xt = xt_ref[...]
    h = h_ref[...]
    c = c_ref[...]
    gates = jnp.dot(xt, WihT_ref[...], preferred_element_type=jnp.float32)
    gates = gates + jnp.dot(h, WhhT_ref[...], preferred_element_type=jnp.float32)
    gates = gates + b_ref[...]
    i = jax.nn.sigmoid(gates[:, 0:H])
    f = jax.nn.sigmoid(gates[:, H:2 * H])
    g = jnp.tanh(gates[:, 2 * H:3 * H])
    o = jax.nn.sigmoid(gates[:, 3 * H:4 * H])
    cn = f * c + i * g
    hn = o * jnp.tanh(cn)
    t = t_ref[0]
    valid = deg_ref[...] > t
    h_out[...] = jnp.where(valid, hn, h)
    c_out[...] = jnp.where(valid, cn, c)


def _update_body(x_ref, hN_ref, WsT_ref, WnT_ref, bs_ref, x_out):
    acc = jnp.dot(x_ref[...], WsT_ref[...], preferred_element_type=jnp.float32)
    acc = acc + jnp.dot(hN_ref[...], WnT_ref[...], preferred_element_type=jnp.float32)
    x_out[...] = jax.nn.relu(acc + bs_ref[...])


def _heads_body(x_ref, W1T_ref, b1_ref, W2T_ref, b2_ref, o_ref, lo_ref):
    x = x_ref[...]
    o_ref[...] = jnp.dot(x, W1T_ref[...], preferred_element_type=jnp.float32) + b1_ref[...]
    lo_ref[...] = jnp.dot(x, W2T_ref[...], preferred_element_type=jnp.float32) + b2_ref[...]


def kernel(h, edge_index, W_ih, W_hh, b_ih, b_hh, W_self, b_self, W_neigh,
           W1, b1, W2, b2):
    N, H = h.shape
    E = edge_index.shape[1]
    L = W_ih.shape[0]
    NUM_OUT = W1.shape[0]

    BN = 2000
    NB = N // BN
    assert NB * BN == N

    src = edge_index[0]
    dst = edge_index[1]
    order = jnp.argsort(dst)
    s_src = src[order]
    deg = jnp.bincount(dst, length=N)
    offsets = jnp.cumsum(deg) - deg
    T = deg.max()
    deg2d = deg[:, None]

    cell = pl.pallas_call(
        _cell_body,
        grid_spec=pltpu.PrefetchScalarGridSpec(
            num_scalar_prefetch=1,
            grid=(NB,),
            in_specs=[
                pl.BlockSpec((BN, H), lambda i, t: (i, 0)),
                pl.BlockSpec((BN, H), lambda i, t: (i, 0)),
                pl.BlockSpec((BN, H), lambda i, t: (i, 0)),
                pl.BlockSpec((BN, 1), lambda i, t: (i, 0)),
                pl.BlockSpec((H, 4 * H), lambda i, t: (0, 0)),
                pl.BlockSpec((H, 4 * H), lambda i, t: (0, 0)),
                pl.BlockSpec((1, 4 * H), lambda i, t: (0, 0)),
            ],
            out_specs=[
                pl.BlockSpec((BN, H), lambda i, t: (i, 0)),
                pl.BlockSpec((BN, H), lambda i, t: (i, 0)),
            ],
        ),
        out_shape=[
            jax.ShapeDtypeStruct((N, H), jnp.float32),
            jax.ShapeDtypeStruct((N, H), jnp.float32),
        ],
        input_output_aliases={2: 0, 3: 1},
    )

    update = pl.pallas_call(
        _update_body,
        grid=(NB,),
        in_specs=[
            pl.BlockSpec((BN, H), lambda i: (i, 0)),
            pl.BlockSpec((BN, H), lambda i: (i, 0)),
            pl.BlockSpec((H, H), lambda i: (0, 0)),
            pl.BlockSpec((H, H), lambda i: (0, 0)),
            pl.BlockSpec((1, H), lambda i: (0, 0)),
        ],
        out_specs=pl.BlockSpec((BN, H), lambda i: (i, 0)),
        out_shape=jax.ShapeDtypeStruct((N, H), jnp.float32),
    )

    heads = pl.pallas_call(
        _heads_body,
        grid=(NB,),
        in_specs=[
            pl.BlockSpec((BN, H), lambda i: (i, 0)),
            pl.BlockSpec((H, NUM_OUT), lambda i: (0, 0)),
            pl.BlockSpec((1, NUM_OUT), lambda i: (0, 0)),
            pl.BlockSpec((H, 1), lambda i: (0, 0)),
            pl.BlockSpec((1, 1), lambda i: (0, 0)),
        ],
        out_specs=[
            pl.BlockSpec((BN, NUM_OUT), lambda i: (i, 0)),
            pl.BlockSpec((BN, 1), lambda i: (i, 0)),
        ],
        out_shape=[
            jax.ShapeDtypeStruct((N, NUM_OUT), jnp.float32),
            jax.ShapeDtypeStruct((N, 1), jnp.float32),
        ],
    )

    x = h
    for li in range(L):
        WihT = W_ih[li].T
        WhhT = W_hh[li].T
        bsum = (b_ih[li] + b_hh[li])[None, :]
        WsT = W_self[li].T
        WnT = W_neigh[li].T
        bs = b_self[li][None, :]

        def body(state):
            hs, cs, t = state
            idx = s_src[jnp.clip(offsets + t, 0, E - 1)]
            xt = x[idx]
            hs, cs = cell(jnp.array([t], jnp.int32), xt, hs, cs, deg2d,
                          WihT, WhhT, bsum)
            return (hs, cs, t + 1)

        def cond(state):
            return state[2] < T

        h0 = jnp.zeros((N, H), jnp.float32)
        c0 = jnp.zeros((N, H), jnp.float32)
        hN, _, _ = jax.lax.while_loop(cond, body, (h0, c0, jnp.int32(0)))
        x = update(x, hN, WsT, WnT, bs)

    o, lo = heads(x, W1.T, b1[None, :], W2.T, b2[None, :])
    return (o, x, lo)
